# baseline (speedup 1.0000x reference)
import functools

import jax
import jax.numpy as jnp
from jax import lax
from jax.experimental import pallas as pl
from jax.experimental.pallas import tpu as pltpu
from jax.experimental.pallas import tpu_sc as plsc

E = 64
D = 768
H = 768
N = 8192
C = 128
EC = E * C

RB = 256
NB = N // RB

NC = 2
NS = 16
NW = NC * NS
TPW = N // NW
CH = 128
NCHUNK = TPW // CH


def _gate_body(x_ref, wg_ref, slotg_ref, gk_ref, slots_ref, carry_ref):
    i = pl.program_id(0)

    @pl.when(i == 0)
    def _():
        carry_ref[...] = jnp.zeros_like(carry_ref)

    lg = lax.dot_general(wg_ref[...], x_ref[...], (((0,), (1,)), ((), ())),
                         preferred_element_type=jnp.float32)
    m = jnp.max(lg, axis=0, keepdims=True)
    gate = 1.0 / jnp.sum(jnp.exp(lg - m), axis=0, keepdims=True)
    iota_e = lax.broadcasted_iota(jnp.int32, (E, RB), 0)
    top1 = jnp.min(jnp.where(lg == m, iota_e, E), axis=0, keepdims=True)
    onehot = (iota_e == top1).astype(jnp.float32)
    ii = lax.broadcasted_iota(jnp.int32, (RB, RB), 0)
    jj = lax.broadcasted_iota(jnp.int32, (RB, RB), 1)
    striu = (ii < jj).astype(jnp.float32)
    before = jnp.dot(onehot, striu, preferred_element_type=jnp.float32)
    carry = carry_ref[...]
    pos = jnp.sum((before + carry) * onehot, axis=0,
                  keepdims=True).astype(jnp.int32)
    ones_rb = jnp.full((RB, 1), 1.0, dtype=jnp.float32)
    carry_ref[...] = carry + jnp.dot(onehot, ones_rb,
                                     preferred_element_type=jnp.float32)
    keep = pos < C
    posc = jnp.minimum(pos, C - 1)
    slotg = top1 * C + posc
    tok = i * RB + lax.broadcasted_iota(jnp.int32, (1, RB), 1)
    slotg_ref[...] = slotg.reshape(1, 1, RB)
    gk_ref[...] = (gate * keep.astype(jnp.float32)).reshape(1, 1, RB)
    slots_ref[...] = jnp.where(keep, slotg, EC + tok).reshape(1, 1, RB)


def _gate(x, wg):
    return pl.pallas_call(
        _gate_body,
        grid=(NB,),
        in_specs=[
            pl.BlockSpec((RB, D), lambda i: (i, 0)),
            pl.BlockSpec((D, E), lambda i: (0, 0)),
        ],
        out_specs=[
            pl.BlockSpec((1, 1, RB), lambda i: (i, 0, 0)),
            pl.BlockSpec((1, 1, RB), lambda i: (i, 0, 0)),
            pl.BlockSpec((1, 1, RB), lambda i: (i, 0, 0)),
        ],
        out_shape=[
            jax.ShapeDtypeStruct((NB, 1, RB), jnp.int32),
            jax.ShapeDtypeStruct((NB, 1, RB), jnp.float32),
            jax.ShapeDtypeStruct((NB, 1, RB), jnp.int32),
        ],
        scratch_shapes=[pltpu.VMEM((E, 1), jnp.float32)],
    )(x, wg)


@functools.cache
def _dispatch_kernel():
    mesh = plsc.VectorSubcoreMesh(core_axis_name="c", subcore_axis_name="s")

    @functools.partial(
        pl.kernel,
        mesh=mesh,
        out_type=jax.ShapeDtypeStruct((EC + N, D), jnp.float32),
        scratch_types=[
            pltpu.VMEM((NCHUNK, CH), jnp.int32),
            pltpu.VMEM((CH, D), jnp.float32),
            pltpu.SemaphoreType.DMA,
        ],
    )
    def _dispatch(x_hbm, slots_hbm, xin_hbm, idx_v, buf_v, sem):
        wid = lax.axis_index("s") * NC + lax.axis_index("c")
        base = wid * TPW
        pltpu.sync_copy(slots_hbm.at[wid], idx_v)
        for j in range(NCHUNK):
            pltpu.sync_copy(x_hbm.at[pl.ds(base + j * CH, CH)], buf_v)
            pltpu.async_copy(buf_v, xin_hbm.at[idx_v.at[j]], sem).wait()

    return _dispatch


EBW2 = 4
W2R = EBW2 * H


def _w2sum_body(w2_ref, out_ref):
    ones = jnp.full((D, 1), 1.0, dtype=jnp.float32)
    out_ref[...] = jnp.dot(w2_ref[...], ones, preferred_element_type=jnp.float32)


def _w2sum(w2flat):
    return pl.pallas_call(
        _w2sum_body,
        grid=(E // EBW2,),
        in_specs=[pl.BlockSpec((W2R, D), lambda e: (e, 0))],
        out_specs=pl.BlockSpec((W2R, 1), lambda e: (e, 0)),
        out_shape=jax.ShapeDtypeStruct((E * H, 1), jnp.float32),
    )(w2flat)


EB = 4


def _ffn_body(xin_ref, w1_ref, b1_ref, w2s_ref, b2_ref, out_ref):
    for k in range(EB):
        h = jnp.dot(xin_ref[pl.ds(k * C, C), :], w1_ref[k],
                    preferred_element_type=jnp.float32)
        h = jnp.maximum(h + b1_ref[k], 0.0)
        b2s = jnp.sum(b2_ref[k])
        out_ref[pl.ds(k * C, C), :] = (
            jnp.dot(h, w2s_ref[k], preferred_element_type=jnp.float32) + b2s)


def _ffn(xin, w1, b1, w2s, b2):
    return pl.pallas_call(
        _ffn_body,
        grid=(E // EB,),
        in_specs=[
            pl.BlockSpec((EB * C, D), lambda e: (e, 0)),
            pl.BlockSpec((EB, D, H), lambda e: (e, 0, 0)),
            pl.BlockSpec((EB, 1, H), lambda e: (e, 0, 0)),
            pl.BlockSpec((EB, H, 1), lambda e: (e, 0, 0)),
            pl.BlockSpec((EB, 1, D), lambda e: (e, 0, 0)),
        ],
        out_specs=pl.BlockSpec((EB * C, 1), lambda e: (e, 0)),
        out_shape=jax.ShapeDtypeStruct((EC, 1), jnp.float32),
    )(xin, w1, b1, w2s, b2)


@functools.cache
def _combine_kernel():
    mesh = plsc.VectorSubcoreMesh(core_axis_name="c", subcore_axis_name="s")

    @functools.partial(
        pl.kernel,
        mesh=mesh,
        out_type=jax.ShapeDtypeStruct((N,), jnp.float32),
        compiler_params=pltpu.CompilerParams(needs_layout_passes=False),
        scratch_types=[
            pltpu.VMEM((EC,), jnp.float32),
            pltpu.VMEM((TPW,), jnp.int32),
            pltpu.VMEM((TPW,), jnp.float32),
            pltpu.VMEM((TPW,), jnp.float32),
        ],
    )
    def _combine(table_hbm, slotg_hbm, gk_hbm, out_hbm, table_v, idx_v, gk_v, s_v):
        wid = lax.axis_index("s") * NC + lax.axis_index("c")
        base = wid * TPW
        pltpu.sync_copy(table_hbm, table_v)
        pltpu.sync_copy(slotg_hbm.at[wid], idx_v)
        pltpu.sync_copy(gk_hbm.at[wid], gk_v)
        for i in range(TPW // 16):
            sl = pl.ds(i * 16, 16)
            vals = plsc.load_gather(table_v, [idx_v[sl]])
            s_v[sl] = vals * gk_v[sl]
        pltpu.sync_copy(s_v, out_hbm.at[pl.ds(base, TPW)])

    return _combine


def _lsm_body(s_ref, o_ref):
    s = s_ref[...]
    z = s - jnp.max(s)
    o_ref[...] = z - jnp.log(jnp.sum(jnp.exp(z)))


def _lsm(s):
    return pl.pallas_call(
        _lsm_body,
        out_shape=jax.ShapeDtypeStruct((E, C), jnp.float32),
    )(s)


def kernel(input, Wg, W1, b1, W2, b2):
    x = input.reshape(N, D)
    slotg, gk, slots = _gate(x, Wg)
    xin = _dispatch_kernel()(x, slots.reshape(NW, NCHUNK, CH))
    w2s = _w2sum(W2.reshape(E * H, D)).reshape(E, H, 1)
    ss = _ffn(xin, W1, b1.reshape(E, 1, H), w2s, b2.reshape(E, 1, D))
    s = _combine_kernel()(ss.reshape(EC), slotg.reshape(NW, TPW), gk.reshape(NW, TPW))
    return _lsm(s.reshape(E, C)).reshape(1, N)


# speedup vs baseline: 1.1279x; 1.1279x over previous
import functools

import jax
import jax.numpy as jnp
from jax import lax
from jax.experimental import pallas as pl
from jax.experimental.pallas import tpu as pltpu
from jax.experimental.pallas import tpu_sc as plsc

E = 64
D = 768
H = 768
N = 8192
C = 128
EC = E * C

RB = 512
NB = N // RB

NC = 2
NS = 16
NW = NC * NS
TPW = N // NW
CH = 128
NCHUNK = TPW // CH


def _gate_body(x_ref, wg_ref, slotg_ref, gk_ref, slots_ref, carry_ref):
    i = pl.program_id(0)

    @pl.when(i == 0)
    def _():
        carry_ref[...] = jnp.zeros_like(carry_ref)

    lg = lax.dot_general(wg_ref[...], x_ref[...], (((0,), (1,)), ((), ())),
                         preferred_element_type=jnp.float32)
    m = jnp.max(lg, axis=0, keepdims=True)
    gate = 1.0 / jnp.sum(jnp.exp(lg - m), axis=0, keepdims=True)
    iota_e = lax.broadcasted_iota(jnp.int32, (E, RB), 0)
    top1 = jnp.min(jnp.where(lg == m, iota_e, E), axis=0, keepdims=True)
    onehot = (iota_e == top1).astype(jnp.float32)
    ii = lax.broadcasted_iota(jnp.int32, (RB, RB), 0)
    jj = lax.broadcasted_iota(jnp.int32, (RB, RB), 1)
    striu = (ii < jj).astype(jnp.float32)
    before = jnp.dot(onehot, striu, preferred_element_type=jnp.float32)
    carry = carry_ref[...]
    pos = jnp.sum((before + carry) * onehot, axis=0,
                  keepdims=True).astype(jnp.int32)
    ones_rb = jnp.full((RB, 1), 1.0, dtype=jnp.float32)
    carry_ref[...] = carry + jnp.dot(onehot, ones_rb,
                                     preferred_element_type=jnp.float32)
    keep = pos < C
    posc = jnp.minimum(pos, C - 1)
    slotg = top1 * C + posc
    tok = i * RB + lax.broadcasted_iota(jnp.int32, (1, RB), 1)
    slotg_ref[...] = slotg.reshape(1, 1, RB)
    gk_ref[...] = (gate * keep.astype(jnp.float32)).reshape(1, 1, RB)
    slots_ref[...] = jnp.where(keep, slotg, EC + tok).reshape(1, 1, RB)


def _gate(x, wg):
    return pl.pallas_call(
        _gate_body,
        grid=(NB,),
        in_specs=[
            pl.BlockSpec((RB, D), lambda i: (i, 0)),
            pl.BlockSpec((D, E), lambda i: (0, 0)),
        ],
        out_specs=[
            pl.BlockSpec((1, 1, RB), lambda i: (i, 0, 0)),
            pl.BlockSpec((1, 1, RB), lambda i: (i, 0, 0)),
            pl.BlockSpec((1, 1, RB), lambda i: (i, 0, 0)),
        ],
        out_shape=[
            jax.ShapeDtypeStruct((NB, 1, RB), jnp.int32),
            jax.ShapeDtypeStruct((NB, 1, RB), jnp.float32),
            jax.ShapeDtypeStruct((NB, 1, RB), jnp.int32),
        ],
        scratch_shapes=[pltpu.VMEM((E, 1), jnp.float32)],
    )(x, wg)


@functools.cache
def _dispatch_kernel():
    mesh = plsc.VectorSubcoreMesh(core_axis_name="c", subcore_axis_name="s")

    @functools.partial(
        pl.kernel,
        mesh=mesh,
        out_type=jax.ShapeDtypeStruct((EC + N, D), jnp.float32),
        scratch_types=[
            pltpu.VMEM((NCHUNK, CH), jnp.int32),
            pltpu.VMEM((CH, D), jnp.float32),
            pltpu.SemaphoreType.DMA,
        ],
    )
    def _dispatch(x_hbm, slots_hbm, xin_hbm, idx_v, buf_v, sem):
        wid = lax.axis_index("s") * NC + lax.axis_index("c")
        base = wid * TPW
        pltpu.sync_copy(slots_hbm.at[wid], idx_v)
        for j in range(NCHUNK):
            pltpu.sync_copy(x_hbm.at[pl.ds(base + j * CH, CH)], buf_v)
            pltpu.async_copy(buf_v, xin_hbm.at[idx_v.at[j]], sem).wait()

    return _dispatch


EB = 4


def _ffn_body(xin_ref, w1_ref, b1_ref, w2_ref, b2_ref, out_ref):
    ones = jnp.full((D, 1), 1.0, dtype=jnp.float32)
    for k in range(EB):
        h = jnp.dot(xin_ref[pl.ds(k * C, C), :], w1_ref[k],
                    preferred_element_type=jnp.float32)
        h = jnp.maximum(h + b1_ref[k], 0.0)
        w2s = jnp.dot(w2_ref[k], ones, preferred_element_type=jnp.float32)
        b2s = jnp.sum(b2_ref[k])
        out_ref[pl.ds(k * C, C), :] = (
            jnp.dot(h, w2s, preferred_element_type=jnp.float32) + b2s)


def _ffn(xin, w1, b1, w2, b2):
    return pl.pallas_call(
        _ffn_body,
        grid=(E // EB,),
        in_specs=[
            pl.BlockSpec((EB * C, D), lambda e: (e, 0)),
            pl.BlockSpec((EB, D, H), lambda e: (e, 0, 0)),
            pl.BlockSpec((EB, 1, H), lambda e: (e, 0, 0)),
            pl.BlockSpec((EB, H, D), lambda e: (e, 0, 0)),
            pl.BlockSpec((EB, 1, D), lambda e: (e, 0, 0)),
        ],
        out_specs=pl.BlockSpec((EB * C, 1), lambda e: (e, 0)),
        out_shape=jax.ShapeDtypeStruct((EC, 1), jnp.float32),
    )(xin, w1, b1, w2, b2)


@functools.cache
def _combine_kernel():
    mesh = plsc.VectorSubcoreMesh(core_axis_name="c", subcore_axis_name="s")

    @functools.partial(
        pl.kernel,
        mesh=mesh,
        out_type=jax.ShapeDtypeStruct((N,), jnp.float32),
        compiler_params=pltpu.CompilerParams(needs_layout_passes=False),
        scratch_types=[
            pltpu.VMEM((EC,), jnp.float32),
            pltpu.VMEM((TPW,), jnp.int32),
            pltpu.VMEM((TPW,), jnp.float32),
            pltpu.VMEM((TPW,), jnp.float32),
        ],
    )
    def _combine(table_hbm, slotg_hbm, gk_hbm, out_hbm, table_v, idx_v, gk_v, s_v):
        wid = lax.axis_index("s") * NC + lax.axis_index("c")
        base = wid * TPW
        pltpu.sync_copy(table_hbm, table_v)
        pltpu.sync_copy(slotg_hbm.at[wid], idx_v)
        pltpu.sync_copy(gk_hbm.at[wid], gk_v)
        for i in range(TPW // 16):
            sl = pl.ds(i * 16, 16)
            vals = plsc.load_gather(table_v, [idx_v[sl]])
            s_v[sl] = vals * gk_v[sl]
        pltpu.sync_copy(s_v, out_hbm.at[pl.ds(base, TPW)])

    return _combine


def _lsm_body(s_ref, o_ref):
    s = s_ref[...]
    z = s - jnp.max(s)
    o_ref[...] = z - jnp.log(jnp.sum(jnp.exp(z)))


def _lsm(s):
    return pl.pallas_call(
        _lsm_body,
        out_shape=jax.ShapeDtypeStruct((E, C), jnp.float32),
    )(s)


def kernel(input, Wg, W1, b1, W2, b2):
    x = input.reshape(N, D)
    slotg, gk, slots = _gate(x, Wg)
    xin = _dispatch_kernel()(x, slots.reshape(NW, NCHUNK, CH))
    ss = _ffn(xin, W1, b1.reshape(E, 1, H), W2, b2.reshape(E, 1, D))
    s = _combine_kernel()(ss.reshape(EC), slotg.reshape(NW, TPW), gk.reshape(NW, TPW))
    return _lsm(s.reshape(E, C)).reshape(1, N)


# speedup vs baseline: 1.1561x; 1.0251x over previous
import functools

import jax
import jax.numpy as jnp
from jax import lax
from jax.experimental import pallas as pl
from jax.experimental.pallas import tpu as pltpu
from jax.experimental.pallas import tpu_sc as plsc

E = 64
D = 768
H = 768
N = 8192
C = 128
EC = E * C

RB = 1024
NB = N // RB

NC = 2
NS = 16
NW = NC * NS
TPW = N // NW
CH = 128
NCHUNK = TPW // CH


def _gate_body(x_ref, wg_ref, slotg_ref, gk_ref, slots_ref, carry_ref):
    i = pl.program_id(0)

    @pl.when(i == 0)
    def _():
        carry_ref[...] = jnp.zeros_like(carry_ref)

    lg = lax.dot_general(wg_ref[...], x_ref[...], (((0,), (1,)), ((), ())),
                         preferred_element_type=jnp.float32)
    m = jnp.max(lg, axis=0, keepdims=True)
    gate = 1.0 / jnp.sum(jnp.exp(lg - m), axis=0, keepdims=True)
    iota_e = lax.broadcasted_iota(jnp.int32, (E, RB), 0)
    top1 = jnp.min(jnp.where(lg == m, iota_e, E), axis=0, keepdims=True)
    onehot = (iota_e == top1).astype(jnp.float32)
    ii = lax.broadcasted_iota(jnp.int32, (RB, RB), 0)
    jj = lax.broadcasted_iota(jnp.int32, (RB, RB), 1)
    striu = (ii < jj).astype(jnp.float32)
    before = jnp.dot(onehot, striu, preferred_element_type=jnp.float32)
    carry = carry_ref[...]
    pos = jnp.sum((before + carry) * onehot, axis=0,
                  keepdims=True).astype(jnp.int32)
    ones_rb = jnp.full((RB, 1), 1.0, dtype=jnp.float32)
    carry_ref[...] = carry + jnp.dot(onehot, ones_rb,
                                     preferred_element_type=jnp.float32)
    keep = pos < C
    posc = jnp.minimum(pos, C - 1)
    slotg = top1 * C + posc
    tok = i * RB + lax.broadcasted_iota(jnp.int32, (1, RB), 1)
    slotg_ref[...] = slotg.reshape(1, 1, RB)
    gk_ref[...] = (gate * keep.astype(jnp.float32)).reshape(1, 1, RB)
    slots_ref[...] = jnp.where(keep, slotg, EC + tok).reshape(1, 1, RB)


def _gate(x, wg):
    return pl.pallas_call(
        _gate_body,
        grid=(NB,),
        in_specs=[
            pl.BlockSpec((RB, D), lambda i: (i, 0)),
            pl.BlockSpec((D, E), lambda i: (0, 0)),
        ],
        out_specs=[
            pl.BlockSpec((1, 1, RB), lambda i: (i, 0, 0)),
            pl.BlockSpec((1, 1, RB), lambda i: (i, 0, 0)),
            pl.BlockSpec((1, 1, RB), lambda i: (i, 0, 0)),
        ],
        out_shape=[
            jax.ShapeDtypeStruct((NB, 1, RB), jnp.int32),
            jax.ShapeDtypeStruct((NB, 1, RB), jnp.float32),
            jax.ShapeDtypeStruct((NB, 1, RB), jnp.int32),
        ],
        scratch_shapes=[pltpu.VMEM((E, 1), jnp.float32)],
    )(x, wg)


@functools.cache
def _dispatch_kernel():
    mesh = plsc.VectorSubcoreMesh(core_axis_name="c", subcore_axis_name="s")

    @functools.partial(
        pl.kernel,
        mesh=mesh,
        out_type=jax.ShapeDtypeStruct((EC + N, D), jnp.float32),
        scratch_types=[
            pltpu.VMEM((NCHUNK, CH), jnp.int32),
            pltpu.VMEM((CH, D), jnp.float32),
            pltpu.SemaphoreType.DMA,
        ],
    )
    def _dispatch(x_hbm, slots_hbm, xin_hbm, idx_v, buf_v, sem):
        wid = lax.axis_index("s") * NC + lax.axis_index("c")
        base = wid * TPW
        pltpu.sync_copy(slots_hbm.at[wid], idx_v)
        for j in range(NCHUNK):
            pltpu.sync_copy(x_hbm.at[pl.ds(base + j * CH, CH)], buf_v)
            pltpu.async_copy(buf_v, xin_hbm.at[idx_v.at[j]], sem).wait()

    return _dispatch


EB = 4


def _ffn_body(xin_ref, w1_ref, b1_ref, w2_ref, b2_ref, out_ref):
    ones = jnp.full((D, 1), 1.0, dtype=jnp.float32)
    for k in range(EB):
        h = jnp.dot(xin_ref[pl.ds(k * C, C), :], w1_ref[k],
                    preferred_element_type=jnp.float32)
        h = jnp.maximum(h + b1_ref[k], 0.0)
        w2s = jnp.dot(w2_ref[k], ones, preferred_element_type=jnp.float32)
        b2s = jnp.sum(b2_ref[k])
        out_ref[pl.ds(k * C, C), :] = (
            jnp.dot(h, w2s, preferred_element_type=jnp.float32) + b2s)


def _ffn(xin, w1, b1, w2, b2):
    return pl.pallas_call(
        _ffn_body,
        grid=(E // EB,),
        in_specs=[
            pl.BlockSpec((EB * C, D), lambda e: (e, 0)),
            pl.BlockSpec((EB, D, H), lambda e: (e, 0, 0)),
            pl.BlockSpec((EB, 1, H), lambda e: (e, 0, 0)),
            pl.BlockSpec((EB, H, D), lambda e: (e, 0, 0)),
            pl.BlockSpec((EB, 1, D), lambda e: (e, 0, 0)),
        ],
        out_specs=pl.BlockSpec((EB * C, 1), lambda e: (e, 0)),
        out_shape=jax.ShapeDtypeStruct((EC, 1), jnp.float32),
    )(xin, w1, b1, w2, b2)


@functools.cache
def _combine_kernel():
    mesh = plsc.VectorSubcoreMesh(core_axis_name="c", subcore_axis_name="s")

    @functools.partial(
        pl.kernel,
        mesh=mesh,
        out_type=jax.ShapeDtypeStruct((N,), jnp.float32),
        compiler_params=pltpu.CompilerParams(needs_layout_passes=False),
        scratch_types=[
            pltpu.VMEM((EC,), jnp.float32),
            pltpu.VMEM((TPW,), jnp.int32),
            pltpu.VMEM((TPW,), jnp.float32),
            pltpu.VMEM((TPW,), jnp.float32),
        ],
    )
    def _combine(table_hbm, slotg_hbm, gk_hbm, out_hbm, table_v, idx_v, gk_v, s_v):
        wid = lax.axis_index("s") * NC + lax.axis_index("c")
        base = wid * TPW
        pltpu.sync_copy(table_hbm, table_v)
        pltpu.sync_copy(slotg_hbm.at[wid], idx_v)
        pltpu.sync_copy(gk_hbm.at[wid], gk_v)
        for i in range(TPW // 16):
            sl = pl.ds(i * 16, 16)
            vals = plsc.load_gather(table_v, [idx_v[sl]])
            s_v[sl] = vals * gk_v[sl]
        pltpu.sync_copy(s_v, out_hbm.at[pl.ds(base, TPW)])

    return _combine


def _lsm_body(s_ref, o_ref):
    s = s_ref[...]
    z = s - jnp.max(s)
    o_ref[...] = z - jnp.log(jnp.sum(jnp.exp(z)))


def _lsm(s):
    return pl.pallas_call(
        _lsm_body,
        out_shape=jax.ShapeDtypeStruct((E, C), jnp.float32),
    )(s)


def kernel(input, Wg, W1, b1, W2, b2):
    x = input.reshape(N, D)
    slotg, gk, slots = _gate(x, Wg)
    xin = _dispatch_kernel()(x, slots.reshape(NW, NCHUNK, CH))
    ss = _ffn(xin, W1, b1.reshape(E, 1, H), W2, b2.reshape(E, 1, D))
    s = _combine_kernel()(ss.reshape(EC), slotg.reshape(NW, TPW), gk.reshape(NW, TPW))
    return _lsm(s.reshape(E, C)).reshape(1, N)


# speedup vs baseline: 1.1585x; 1.0020x over previous
import functools

import jax
import jax.numpy as jnp
from jax import lax
from jax.experimental import pallas as pl
from jax.experimental.pallas import tpu as pltpu
from jax.experimental.pallas import tpu_sc as plsc

E = 64
D = 768
H = 768
N = 8192
C = 128
EC = E * C

RB = 1024
NB = N // RB

NC = 2
NS = 16
NW = NC * NS
TPW = N // NW
CH = 128
NCHUNK = TPW // CH


def _gate_body(x_ref, wg_ref, slotg_ref, gk_ref, slots_ref, carry_ref):
    i = pl.program_id(0)

    @pl.when(i == 0)
    def _():
        carry_ref[...] = jnp.zeros_like(carry_ref)

    lg = lax.dot_general(wg_ref[...], x_ref[...], (((0,), (1,)), ((), ())),
                         preferred_element_type=jnp.float32)
    m = jnp.max(lg, axis=0, keepdims=True)
    gate = 1.0 / jnp.sum(jnp.exp(lg - m), axis=0, keepdims=True)
    iota_e = lax.broadcasted_iota(jnp.int32, (E, RB), 0)
    top1 = jnp.min(jnp.where(lg == m, iota_e, E), axis=0, keepdims=True)
    onehot = (iota_e == top1).astype(jnp.float32)
    ii = lax.broadcasted_iota(jnp.int32, (RB, RB), 0)
    jj = lax.broadcasted_iota(jnp.int32, (RB, RB), 1)
    striu = (ii < jj).astype(jnp.float32)
    before = jnp.dot(onehot, striu, preferred_element_type=jnp.float32)
    carry = carry_ref[...]
    pos = jnp.sum((before + carry) * onehot, axis=0,
                  keepdims=True).astype(jnp.int32)
    ones_rb = jnp.full((RB, 1), 1.0, dtype=jnp.float32)
    carry_ref[...] = carry + jnp.dot(onehot, ones_rb,
                                     preferred_element_type=jnp.float32)
    keep = pos < C
    posc = jnp.minimum(pos, C - 1)
    slotg = top1 * C + posc
    tok = i * RB + lax.broadcasted_iota(jnp.int32, (1, RB), 1)
    slotg_ref[...] = slotg.reshape(1, 1, RB)
    gk_ref[...] = (gate * keep.astype(jnp.float32)).reshape(1, 1, RB)
    slots_ref[...] = jnp.where(keep, slotg, EC + tok).reshape(1, 1, RB)


def _gate(x, wg):
    return pl.pallas_call(
        _gate_body,
        grid=(NB,),
        in_specs=[
            pl.BlockSpec((RB, D), lambda i: (i, 0)),
            pl.BlockSpec((D, E), lambda i: (0, 0)),
        ],
        out_specs=[
            pl.BlockSpec((1, 1, RB), lambda i: (i, 0, 0)),
            pl.BlockSpec((1, 1, RB), lambda i: (i, 0, 0)),
            pl.BlockSpec((1, 1, RB), lambda i: (i, 0, 0)),
        ],
        out_shape=[
            jax.ShapeDtypeStruct((NB, 1, RB), jnp.int32),
            jax.ShapeDtypeStruct((NB, 1, RB), jnp.float32),
            jax.ShapeDtypeStruct((NB, 1, RB), jnp.int32),
        ],
        scratch_shapes=[pltpu.VMEM((E, 1), jnp.float32)],
        compiler_params=pltpu.CompilerParams(skip_device_barrier=True),
    )(x, wg)


@functools.cache
def _dispatch_kernel():
    mesh = plsc.VectorSubcoreMesh(core_axis_name="c", subcore_axis_name="s")

    @functools.partial(
        pl.kernel,
        mesh=mesh,
        out_type=jax.ShapeDtypeStruct((EC + N, D), jnp.float32),
        scratch_types=[
            pltpu.VMEM((NCHUNK, CH), jnp.int32),
            pltpu.VMEM((CH, D), jnp.float32),
            pltpu.SemaphoreType.DMA,
        ],
    )
    def _dispatch(x_hbm, slots_hbm, xin_hbm, idx_v, buf_v, sem):
        wid = lax.axis_index("s") * NC + lax.axis_index("c")
        base = wid * TPW
        pltpu.sync_copy(slots_hbm.at[wid], idx_v)
        for j in range(NCHUNK):
            pltpu.sync_copy(x_hbm.at[pl.ds(base + j * CH, CH)], buf_v)
            pltpu.async_copy(buf_v, xin_hbm.at[idx_v.at[j]], sem).wait()

    return _dispatch


EB = 4


def _ffn_body(xin_ref, w1_ref, b1_ref, w2_ref, b2_ref, out_ref):
    ones = jnp.full((D, 1), 1.0, dtype=jnp.float32)
    for k in range(EB):
        h = jnp.dot(xin_ref[pl.ds(k * C, C), :], w1_ref[k],
                    preferred_element_type=jnp.float32)
        h = jnp.maximum(h + b1_ref[k], 0.0)
        w2s = jnp.dot(w2_ref[k], ones, preferred_element_type=jnp.float32)
        b2s = jnp.sum(b2_ref[k])
        out_ref[pl.ds(k * C, C), :] = (
            jnp.dot(h, w2s, preferred_element_type=jnp.float32) + b2s)


def _ffn(xin, w1, b1, w2, b2):
    return pl.pallas_call(
        _ffn_body,
        grid=(E // EB,),
        in_specs=[
            pl.BlockSpec((EB * C, D), lambda e: (e, 0)),
            pl.BlockSpec((EB, D, H), lambda e: (e, 0, 0)),
            pl.BlockSpec((EB, 1, H), lambda e: (e, 0, 0)),
            pl.BlockSpec((EB, H, D), lambda e: (e, 0, 0)),
            pl.BlockSpec((EB, 1, D), lambda e: (e, 0, 0)),
        ],
        out_specs=pl.BlockSpec((EB * C, 1), lambda e: (e, 0)),
        out_shape=jax.ShapeDtypeStruct((EC, 1), jnp.float32),
        compiler_params=pltpu.CompilerParams(skip_device_barrier=True),
    )(xin, w1, b1, w2, b2)


@functools.cache
def _combine_kernel():
    mesh = plsc.VectorSubcoreMesh(core_axis_name="c", subcore_axis_name="s")

    @functools.partial(
        pl.kernel,
        mesh=mesh,
        out_type=jax.ShapeDtypeStruct((N,), jnp.float32),
        compiler_params=pltpu.CompilerParams(needs_layout_passes=False),
        scratch_types=[
            pltpu.VMEM((EC,), jnp.float32),
            pltpu.VMEM((TPW,), jnp.int32),
            pltpu.VMEM((TPW,), jnp.float32),
            pltpu.VMEM((TPW,), jnp.float32),
        ],
    )
    def _combine(table_hbm, slotg_hbm, gk_hbm, out_hbm, table_v, idx_v, gk_v, s_v):
        wid = lax.axis_index("s") * NC + lax.axis_index("c")
        base = wid * TPW
        pltpu.sync_copy(table_hbm, table_v)
        pltpu.sync_copy(slotg_hbm.at[wid], idx_v)
        pltpu.sync_copy(gk_hbm.at[wid], gk_v)
        for i in range(TPW // 16):
            sl = pl.ds(i * 16, 16)
            vals = plsc.load_gather(table_v, [idx_v[sl]])
            s_v[sl] = vals * gk_v[sl]
        pltpu.sync_copy(s_v, out_hbm.at[pl.ds(base, TPW)])

    return _combine


def _lsm_body(s_ref, o_ref):
    s = s_ref[...]
    z = s - jnp.max(s)
    o_ref[...] = z - jnp.log(jnp.sum(jnp.exp(z)))


def _lsm(s):
    return pl.pallas_call(
        _lsm_body,
        out_shape=jax.ShapeDtypeStruct((E, C), jnp.float32),
        compiler_params=pltpu.CompilerParams(skip_device_barrier=True),
    )(s)


def kernel(input, Wg, W1, b1, W2, b2):
    x = input.reshape(N, D)
    slotg, gk, slots = _gate(x, Wg)
    xin = _dispatch_kernel()(x, slots.reshape(NW, NCHUNK, CH))
    ss = _ffn(xin, W1, b1.reshape(E, 1, H), W2, b2.reshape(E, 1, D))
    s = _combine_kernel()(ss.reshape(EC), slotg.reshape(NW, TPW), gk.reshape(NW, TPW))
    return _lsm(s.reshape(E, C)).reshape(1, N)
